# Initial kernel scaffold; baseline (speedup 1.0000x reference)
import jax
import jax.numpy as jnp
from jax.experimental import pallas as pl


def kernel(inp, src_length, tgt_input, table):
    raise NotImplementedError("write your pallas kernel here")


# speedup vs baseline: 2.9076x; 2.9076x over previous
import functools

import jax
import jax.numpy as jnp
from jax import lax
from jax.experimental import pallas as pl
from jax.experimental.pallas import tpu as pltpu
from jax.experimental.pallas import tpu_sc as plsc

_VOCAB = 32320
_DIM = 1024
_B = 4
_L = 2048
_N = _B * _L
_NC, _NS, _LANES = 2, 16, 16
_NW = _NC * _NS
_RPW = _N // _NW
_C = 32
_NCHUNK = _RPW // _C

_mesh = plsc.VectorSubcoreMesh(
    core_axis_name="c", subcore_axis_name="s",
    num_cores=_NC, num_subcores=_NS)


def _fix_padding_rows(idx_v, rows_v, off):
    def row_body(r, _):
        splat = plsc.load_gather(
            idx_v, [jnp.full((_LANES,), off, jnp.int32) + r])
        scale = jnp.where(splat == 0, 0.0, 1.0)

        def col_body(cc, _):
            seg = rows_v[r, pl.ds(cc * _LANES, _LANES)]
            rows_v[r, pl.ds(cc * _LANES, _LANES)] = seg * scale
            return 0

        lax.fori_loop(0, _DIM // _LANES, col_body, 0)
        return 0

    lax.fori_loop(0, _C, row_body, 0)


@functools.partial(
    pl.kernel,
    out_type=jax.ShapeDtypeStruct((_N, _DIM), jnp.float32),
    mesh=_mesh,
    scratch_types=[
        pltpu.VMEM((_RPW,), jnp.int32),
        pltpu.VMEM((_C, _DIM), jnp.float32),
        pltpu.SemaphoreType.DMA,
    ],
    compiler_params=pltpu.CompilerParams(needs_layout_passes=False),
)
def _embed(idx_hbm, table_hbm, out_hbm, idx_v, rows_v, sem):
    wid = lax.axis_index("s") * _NC + lax.axis_index("c")
    base = wid * _RPW
    pltpu.sync_copy(idx_hbm.at[pl.ds(base, _RPW)], idx_v)

    for t in range(_NCHUNK):
        off = t * _C
        pltpu.async_copy(
            table_hbm.at[idx_v.at[pl.ds(off, _C)]], rows_v, sem).wait()

        z = idx_v[pl.ds(off, _LANES)]
        for g in range(1, _C // _LANES):
            z = jnp.minimum(z, idx_v[pl.ds(off + g * _LANES, _LANES)])
        zs, _ = plsc.sort_key_val(z, z)
        haszero = zs[0] == 0

        @pl.when(haszero)
        def _():
            _fix_padding_rows(idx_v, rows_v, off)

        pltpu.sync_copy(rows_v, out_hbm.at[pl.ds(base + off, _C)])


def kernel(inp, src_length, tgt_input, table):
    idx = tgt_input.reshape(_N).astype(jnp.int32)
    out = _embed(idx, table)
    return (inp, src_length, out.reshape(_B, _L, _DIM))


# speedup vs baseline: 3.1969x; 1.0995x over previous
import functools

import jax
import jax.numpy as jnp
from jax import lax
from jax.experimental import pallas as pl
from jax.experimental.pallas import tpu as pltpu
from jax.experimental.pallas import tpu_sc as plsc

_VOCAB = 32320
_DIM = 1024
_B = 4
_L = 2048
_N = _B * _L
_NC, _NS, _LANES = 2, 16, 16
_NW = _NC * _NS
_RPW = _N // _NW
_C = 32
_NCHUNK = _RPW // _C

_mesh = plsc.VectorSubcoreMesh(
    core_axis_name="c", subcore_axis_name="s",
    num_cores=_NC, num_subcores=_NS)


def _fix_padding_rows(idx_v, rows_v, off):
    def row_body(r, _):
        splat = plsc.load_gather(
            idx_v, [jnp.full((_LANES,), off, jnp.int32) + r])
        scale = jnp.where(splat == 0, 0.0, 1.0)

        def col_body(cc, _):
            seg = rows_v[r, pl.ds(cc * _LANES, _LANES)]
            rows_v[r, pl.ds(cc * _LANES, _LANES)] = seg * scale
            return 0

        lax.fori_loop(0, _DIM // _LANES, col_body, 0)
        return 0

    lax.fori_loop(0, _C, row_body, 0)


def _chunk_has_zero(idx_v, off):
    z = idx_v[pl.ds(off, _LANES)]
    for g in range(1, _C // _LANES):
        z = jnp.minimum(z, idx_v[pl.ds(off + g * _LANES, _LANES)])
    zs, _ = plsc.sort_key_val(z, z)
    return zs[0] == 0


@functools.partial(
    pl.kernel,
    out_type=jax.ShapeDtypeStruct((_N, _DIM), jnp.float32),
    mesh=_mesh,
    scratch_types=[
        pltpu.VMEM((_RPW,), jnp.int32),
        pltpu.VMEM((_C, _DIM), jnp.float32),
        pltpu.VMEM((_C, _DIM), jnp.float32),
        pltpu.SemaphoreType.DMA,
        pltpu.SemaphoreType.DMA,
        pltpu.SemaphoreType.DMA,
        pltpu.SemaphoreType.DMA,
    ],
    compiler_params=pltpu.CompilerParams(needs_layout_passes=False),
)
def _embed(idx_hbm, table_hbm, out_hbm, idx_v, rows_a, rows_b,
           gsem_a, gsem_b, wsem_a, wsem_b):
    wid = lax.axis_index("s") * _NC + lax.axis_index("c")
    base = wid * _RPW
    pltpu.sync_copy(idx_hbm.at[pl.ds(base, _RPW)], idx_v)

    bufs = (rows_a, rows_b)
    gsems = (gsem_a, gsem_b)
    wsems = (wsem_a, wsem_b)

    def gather(t):
        return pltpu.async_copy(
            table_hbm.at[idx_v.at[pl.ds(t * _C, _C)]],
            bufs[t % 2], gsems[t % 2])

    gcopies = [gather(0)]
    wcopies = []
    for t in range(_NCHUNK):
        if t + 1 < _NCHUNK:
            if t - 1 >= 0:
                wcopies[t - 1].wait()
            gcopies.append(gather(t + 1))
        gcopies[t].wait()

        @pl.when(_chunk_has_zero(idx_v, t * _C))
        def _():
            _fix_padding_rows(idx_v, bufs[t % 2], t * _C)

        wcopies.append(pltpu.async_copy(
            bufs[t % 2], out_hbm.at[pl.ds(base + t * _C, _C)],
            wsems[t % 2]))

    wcopies[_NCHUNK - 2].wait()
    wcopies[_NCHUNK - 1].wait()


def kernel(inp, src_length, tgt_input, table):
    idx = tgt_input.reshape(_N).astype(jnp.int32)
    out = _embed(idx, table)
    return (inp, src_length, out.reshape(_B, _L, _DIM))
